# baseline (speedup 1.0000x reference)
import functools

import jax
import jax.numpy as jnp
from jax import lax
from jax.experimental import pallas as pl
from jax.experimental.pallas import tpu as pltpu
from jax.experimental.pallas import tpu_sc as plsc

VOCAB = 1000000
D = 32
B = 4096
L_SEQ = 50
NC = 2
NS = 16
NW = NC * NS
BPW = B // NW

VC = 12288
VSC = NW * VC
VCC = 1024
NSUB = VC // VCC
assert VC % VCC == 0 and VC % 128 == 0 and VCC % 128 == 0


VBLK = 65536
assert VSC % VBLK == 0
VOFF = VSC // VBLK
VGRID = -(-(VOCAB - VSC) // VBLK)


def _matvec_body(w_ref, embt_ref, p_ref):
    p_ref[...] = jnp.dot(w_ref[...], embt_ref[...],
                         preferred_element_type=jnp.float32)[0]


_matvec = pl.pallas_call(
    _matvec_body,
    grid=(VGRID,),
    in_specs=[
        pl.BlockSpec((1, D), lambda i: (0, 0)),
        pl.BlockSpec((D, VBLK), lambda i: (0, i + VOFF)),
    ],
    out_specs=pl.BlockSpec((VBLK,), lambda i: (i,)),
    out_shape=jax.ShapeDtypeStruct((VGRID * VBLK,), jnp.float32),
)


_mesh = plsc.VectorSubcoreMesh(
    core_axis_name="c", subcore_axis_name="s", num_cores=NC, num_subcores=NS)


def _scmv_body(embt_hbm, w_hbm, plo_hbm, row0, row1, w_v, acc_v, sem0, sem1):
    wid = lax.axis_index("s") * NC + lax.axis_index("c")
    vbase = wid * VC

    pltpu.sync_copy(w_hbm, w_v)
    w0 = w_v[pl.ds(0, 16)]
    w1 = w_v[pl.ds(16, 16)]

    def copy(sub, g, buf, sem):
        return pltpu.make_async_copy(
            embt_hbm.at[pl.ds(g * 8, 8), pl.ds(vbase + sub * VCC, VCC)],
            buf.at[pl.ds(g * 8, 8), :], sem)

    def fire(sub, buf, sem):
        for g in range(D // 8):
            copy(sub, g, buf, sem).start()

    def drain(sub, buf, sem):
        for g in range(D // 8):
            copy(sub, g, buf, sem).wait()

    def compute(sub, buf):
        def cc_body(c, carry):
            acc = jnp.zeros((16,), jnp.float32)
            for d in range(D):
                wd = w0[d] if d < 16 else w1[d - 16]
                acc = acc + buf[d, pl.ds(c * 16, 16)] * wd
            acc_v[pl.ds(sub * VCC + c * 16, 16)] = acc
            return carry
        lax.fori_loop(0, VCC // 16, cc_body, 0)

    bufs = (row0, row1)
    sems = (sem0, sem1)
    fire(0, row0, sem0)
    for sub in range(NSUB):
        if sub + 1 < NSUB:
            fire(sub + 1, bufs[(sub + 1) % 2], sems[(sub + 1) % 2])
        drain(sub, bufs[sub % 2], sems[sub % 2])
        compute(sub, bufs[sub % 2])

    pltpu.sync_copy(acc_v, plo_hbm.at[pl.ds(wid * VC, VC)])


_scmv = functools.partial(
    pl.kernel,
    out_type=jax.ShapeDtypeStruct((VSC,), jnp.float32),
    mesh=_mesh,
    compiler_params=pltpu.CompilerParams(use_tc_tiling_on_sc=True),
    scratch_types=[
        pltpu.VMEM((D, VCC), jnp.float32),
        pltpu.VMEM((D, VCC), jnp.float32),
        pltpu.VMEM((32,), jnp.float32),
        pltpu.VMEM((VC,), jnp.float32),
        pltpu.SemaphoreType.DMA,
        pltpu.SemaphoreType.DMA,
    ],
)(_scmv_body)


def _pool_body(xt_hbm, plo_hbm, phi_hbm, wb_hbm, out_hbm,
               idx_v, idxl_v, idxh_v, vall_v, valh_v, wb_v, out_v, sem):
    wid = lax.axis_index("s") * NC + lax.axis_index("c")
    base = wid * BPW

    pltpu.sync_copy(xt_hbm.at[:, pl.ds(base, BPW)], idx_v)
    pltpu.sync_copy(wb_hbm, wb_v)
    bias = wb_v[pl.ds(0, 16)]
    inv_l = jnp.float32(1.0 / L_SEQ)

    split = jnp.full((16,), VSC, jnp.int32)
    split_m1 = jnp.full((16,), VSC - 1, jnp.int32)
    zero = jnp.zeros((16,), jnp.int32)

    for l in range(L_SEQ):
        for j in range(BPW // 16):
            v = idx_v[l, pl.ds(j * 16, 16)]
            idxl_v[l, pl.ds(j * 16, 16)] = jnp.minimum(v, split_m1)
            idxh_v[l, pl.ds(j * 16, 16)] = jnp.maximum(v - split, zero)

    for l in range(L_SEQ):
        pltpu.make_async_copy(plo_hbm.at[idxl_v.at[l]], vall_v.at[l], sem).start()
        pltpu.make_async_copy(phi_hbm.at[idxh_v.at[l]], valh_v.at[l], sem).start()
    for l in range(L_SEQ):
        pltpu.make_async_copy(plo_hbm.at[idxl_v.at[l]], vall_v.at[l], sem).wait()
        pltpu.make_async_copy(phi_hbm.at[idxh_v.at[l]], valh_v.at[l], sem).wait()

    for j in range(BPW // 16):
        acc = jnp.zeros((16,), jnp.float32)
        for l in range(L_SEQ):
            sel = idx_v[l, pl.ds(j * 16, 16)] < split
            acc = acc + jnp.where(sel,
                                  vall_v[l, pl.ds(j * 16, 16)],
                                  valh_v[l, pl.ds(j * 16, 16)])
        out_v[pl.ds(j * 16, 16)] = acc * inv_l + bias

    pltpu.sync_copy(out_v, out_hbm.at[pl.ds(base, BPW)])


_pool = functools.partial(
    pl.kernel,
    out_type=jax.ShapeDtypeStruct((B,), jnp.float32),
    mesh=_mesh,
    compiler_params=pltpu.CompilerParams(use_tc_tiling_on_sc=False),
    scratch_types=[
        pltpu.VMEM((L_SEQ, BPW), jnp.int32),
        pltpu.VMEM((L_SEQ, BPW), jnp.int32),
        pltpu.VMEM((L_SEQ, BPW), jnp.int32),
        pltpu.VMEM((L_SEQ, BPW), jnp.float32),
        pltpu.VMEM((L_SEQ, BPW), jnp.float32),
        pltpu.VMEM((16,), jnp.float32),
        pltpu.VMEM((BPW,), jnp.float32),
        pltpu.SemaphoreType.DMA,
    ],
)(_pool_body)


@jax.jit
def kernel(x, emb, W, b):
    embt = emb.T
    p_lo = _scmv(embt, W.reshape(D))
    p_hi = _matvec(W, embt)
    wb = jnp.broadcast_to(b, (16,))
    out = _pool(x.astype(jnp.int32).T, p_lo, p_hi, wb)
    return out.reshape(B, 1)


# speedup vs baseline: 9.3301x; 9.3301x over previous
import functools

import jax
import jax.numpy as jnp
from jax import lax
from jax.experimental import pallas as pl
from jax.experimental.pallas import tpu as pltpu
from jax.experimental.pallas import tpu_sc as plsc

VOCAB = 1000000
D = 32
B = 4096
L_SEQ = 50
NC = 2
NS = 16
NW = NC * NS
BPW = B // NW


VBLK = 65536
VGRID = -(-VOCAB // VBLK)


def _matvec_body(w_ref, embt_ref, p_ref):
    p_ref[...] = jnp.dot(w_ref[...], embt_ref[...],
                         preferred_element_type=jnp.float32)[0]


_matvec = pl.pallas_call(
    _matvec_body,
    grid=(VGRID,),
    in_specs=[
        pl.BlockSpec((1, D), lambda i: (0, 0)),
        pl.BlockSpec((D, VBLK), lambda i: (0, i)),
    ],
    out_specs=pl.BlockSpec((VBLK,), lambda i: (i,)),
    out_shape=jax.ShapeDtypeStruct((VGRID * VBLK,), jnp.float32),
)


def _pool_body(xt_hbm, p_hbm, wb_hbm, out_hbm, idx_v, val_v, wb_v, out_v, sem):
    wid = lax.axis_index("s") * NC + lax.axis_index("c")
    base = wid * BPW

    pltpu.sync_copy(xt_hbm.at[:, pl.ds(base, BPW)], idx_v)
    pltpu.sync_copy(wb_hbm, wb_v)
    bias = wb_v[pl.ds(0, 16)]
    inv_l = jnp.float32(1.0 / L_SEQ)

    for l in range(L_SEQ):
        pltpu.make_async_copy(p_hbm.at[idx_v.at[l]], val_v.at[l], sem).start()

    acc = [jnp.zeros((16,), jnp.float32) for _ in range(BPW // 16)]
    for l in range(L_SEQ):
        pltpu.make_async_copy(p_hbm.at[idx_v.at[l]], val_v.at[l], sem).wait()
        for j in range(BPW // 16):
            acc[j] = acc[j] + val_v[l, pl.ds(j * 16, 16)]

    for j in range(BPW // 16):
        out_v[pl.ds(j * 16, 16)] = acc[j] * inv_l + bias

    pltpu.sync_copy(out_v, out_hbm.at[pl.ds(base, BPW)])


_mesh = plsc.VectorSubcoreMesh(
    core_axis_name="c", subcore_axis_name="s", num_cores=NC, num_subcores=NS)

_pool = functools.partial(
    pl.kernel,
    out_type=jax.ShapeDtypeStruct((B,), jnp.float32),
    mesh=_mesh,
    compiler_params=pltpu.CompilerParams(use_tc_tiling_on_sc=False),
    scratch_types=[
        pltpu.VMEM((L_SEQ, BPW), jnp.int32),
        pltpu.VMEM((L_SEQ, BPW), jnp.float32),
        pltpu.VMEM((16,), jnp.float32),
        pltpu.VMEM((BPW,), jnp.float32),
        pltpu.SemaphoreType.DMA,
    ],
)(_pool_body)


@jax.jit
def kernel(x, emb, W, b):
    p = _matvec(W, emb.T)
    wb = jnp.broadcast_to(b, (16,))
    out = _pool(x.astype(jnp.int32).T, p, wb)
    return out.reshape(B, 1)
